# baseline (speedup 1.0000x reference)
import functools

import jax
import jax.numpy as jnp
from jax.experimental import pallas as pl
from jax.experimental.pallas import tpu as pltpu

_BLK = 400


def _gcn_body(x_ref, a_ref, w0_ref, w1_ref, b1_ref, w2_ref, b2_ref,
              out_ref, p1_ref, h1_ref, p2_ref):
    p = pl.program_id(0)
    i = pl.program_id(1)

    @pl.when(p == 0)
    def _():
        h0 = jnp.maximum(
            jnp.dot(x_ref[...], w0_ref[...], preferred_element_type=jnp.float32),
            0.0)
        p1_ref[pl.ds(i * _BLK, _BLK), :] = jnp.dot(
            h0, w1_ref[...], preferred_element_type=jnp.float32)
        out_ref[...] = jnp.zeros_like(out_ref)

    @pl.when(p == 1)
    def _():
        h = jnp.dot(a_ref[...], p1_ref[...], preferred_element_type=jnp.float32)
        h = jnp.maximum(h + b1_ref[...], 0.0)
        h1_ref[pl.ds(i * _BLK, _BLK), :] = h
        out_ref[...] = h

    @pl.when((p == 2) & (i == 0))
    def _():
        p2_ref[...] = jnp.dot(h1_ref[...], w2_ref[...],
                              preferred_element_type=jnp.float32)

    @pl.when(p == 2)
    def _():
        out_ref[...] = jnp.dot(
            a_ref[...], p2_ref[...],
            preferred_element_type=jnp.float32) + b2_ref[...]


@jax.jit
def _gcn(X_sparse, A_norm, W0, W1, b1, W2, b2):
    n, v = X_sparse.shape
    hid = W0.shape[1]
    out_dim = W2.shape[1]
    nblk = n // _BLK

    grid = (3, nblk)
    return pl.pallas_call(
        _gcn_body,
        grid=grid,
        in_specs=[
            pl.BlockSpec((_BLK, v), lambda p, i: (jnp.where(p == 0, i, 0), 0)),
            pl.BlockSpec((_BLK, n), lambda p, i: (jnp.where(p == 0, 0, i), 0)),
            pl.BlockSpec((v, hid), lambda p, i: (0, 0)),
            pl.BlockSpec((hid, hid), lambda p, i: (0, 0)),
            pl.BlockSpec((1, hid), lambda p, i: (0, 0)),
            pl.BlockSpec((hid, out_dim), lambda p, i: (0, 0)),
            pl.BlockSpec((1, out_dim), lambda p, i: (0, 0)),
        ],
        out_specs=pl.BlockSpec((_BLK, out_dim), lambda p, i: (i, 0)),
        out_shape=jax.ShapeDtypeStruct((n, out_dim), jnp.float32),
        scratch_shapes=[
            pltpu.VMEM((n, hid), jnp.float32),
            pltpu.VMEM((n, hid), jnp.float32),
            pltpu.VMEM((n, out_dim), jnp.float32),
        ],
        compiler_params=pltpu.CompilerParams(
            dimension_semantics=("arbitrary", "arbitrary"),
        ),
    )(X_sparse, A_norm, W0, W1, b1.reshape(1, -1), W2, b2.reshape(1, -1))


def kernel(X_sparse, A_norm, W0, W1, b1, W2, b2):
    return _gcn(X_sparse, A_norm, W0, W1, b1, W2, b2)


# speedup vs baseline: 1.0597x; 1.0597x over previous
import jax
import jax.numpy as jnp
from jax.experimental import pallas as pl
from jax.experimental.pallas import tpu as pltpu

_BLK = 400
_XBLK = 2000


def _gcn_body(x_ref, a_ref, w0_ref, w1_ref, b1_ref, w2_ref, b2_ref,
              out_ref, p1_ref, p2_ref):
    s = pl.program_id(0)
    nx = p1_ref.shape[0] // _XBLK
    na = p1_ref.shape[0] // _BLK

    @pl.when(s < nx)
    def _():
        h0 = jnp.maximum(
            jnp.dot(x_ref[...], w0_ref[...], preferred_element_type=jnp.float32),
            0.0)
        p1_ref[pl.ds(s * _XBLK, _XBLK), :] = jnp.dot(
            h0, w1_ref[...], preferred_element_type=jnp.float32)

    @pl.when((s >= nx) & (s < nx + na))
    def _():
        h = jnp.dot(a_ref[...], p1_ref[...], preferred_element_type=jnp.float32)
        h = jnp.maximum(h + b1_ref[...], 0.0)
        p2_ref[pl.ds((s - nx) * _BLK, _BLK), :] = jnp.dot(
            h, w2_ref[...], preferred_element_type=jnp.float32)

    @pl.when(s >= nx + na)
    def _():
        out_ref[...] = jnp.dot(
            a_ref[...], p2_ref[...],
            preferred_element_type=jnp.float32) + b2_ref[...]


@jax.jit
def _gcn(X_sparse, A_norm, W0, W1, b1, W2, b2):
    n, v = X_sparse.shape
    hid = W0.shape[1]
    out_dim = W2.shape[1]
    nx = n // _XBLK
    na = n // _BLK

    grid = (nx + 2 * na,)
    return pl.pallas_call(
        _gcn_body,
        grid=grid,
        in_specs=[
            pl.BlockSpec((_XBLK, v), lambda s: (jnp.where(s < nx, s, 0), 0)),
            pl.BlockSpec(
                (_BLK, n),
                lambda s: (jnp.where(s < nx, 0,
                                     jnp.where(s < nx + na, s - nx,
                                               s - nx - na)), 0)),
            pl.BlockSpec((v, hid), lambda s: (0, 0)),
            pl.BlockSpec((hid, hid), lambda s: (0, 0)),
            pl.BlockSpec((1, hid), lambda s: (0, 0)),
            pl.BlockSpec((hid, out_dim), lambda s: (0, 0)),
            pl.BlockSpec((1, out_dim), lambda s: (0, 0)),
        ],
        out_specs=pl.BlockSpec(
            (_BLK, out_dim),
            lambda s: (jnp.where(s < nx + na, 0, s - nx - na), 0)),
        out_shape=jax.ShapeDtypeStruct((n, out_dim), jnp.float32),
        scratch_shapes=[
            pltpu.VMEM((n, hid), jnp.float32),
            pltpu.VMEM((n, out_dim), jnp.float32),
        ],
        compiler_params=pltpu.CompilerParams(
            dimension_semantics=("arbitrary",),
        ),
    )(X_sparse, A_norm, W0, W1, b1.reshape(1, -1), W2, b2.reshape(1, -1))


def kernel(X_sparse, A_norm, W0, W1, b1, W2, b2):
    return _gcn(X_sparse, A_norm, W0, W1, b1, W2, b2)


# speedup vs baseline: 1.0616x; 1.0018x over previous
import jax
import jax.numpy as jnp
from jax.experimental import pallas as pl
from jax.experimental.pallas import tpu as pltpu

_BLK = 400
_XBLK = 2000


def _gcn_body(x_ref, a_ref, w0_ref, w1_ref, b1_ref, w2_ref, b2_ref,
              out_ref, p1_ref, p2_ref):
    s = pl.program_id(0)
    nx = p1_ref.shape[0] // _XBLK
    na = p1_ref.shape[0] // _BLK

    @pl.when(s < nx)
    def _():
        h0 = jnp.maximum(
            jnp.dot(x_ref[...], w0_ref[...], preferred_element_type=jnp.float32),
            0.0)
        p1_ref[pl.ds(s * _XBLK, _XBLK), :] = jnp.dot(
            h0, w1_ref[...],
            preferred_element_type=jnp.float32).astype(jnp.bfloat16)

    @pl.when((s >= nx) & (s < nx + na))
    def _():
        a_bf = a_ref[...].astype(jnp.bfloat16)
        h = jnp.dot(a_bf, p1_ref[...], preferred_element_type=jnp.float32)
        h = jnp.maximum(h + b1_ref[...], 0.0)
        p2_ref[pl.ds((s - nx) * _BLK, _BLK), :] = jnp.dot(
            h, w2_ref[...],
            preferred_element_type=jnp.float32).astype(jnp.bfloat16)

    @pl.when(s >= nx + na)
    def _():
        a_bf = a_ref[...].astype(jnp.bfloat16)
        out_ref[...] = jnp.dot(
            a_bf, p2_ref[...],
            preferred_element_type=jnp.float32) + b2_ref[...]


@jax.jit
def _gcn(X_sparse, A_norm, W0, W1, b1, W2, b2):
    n, v = X_sparse.shape
    hid = W0.shape[1]
    out_dim = W2.shape[1]
    nx = n // _XBLK
    na = n // _BLK

    grid = (nx + 2 * na,)
    return pl.pallas_call(
        _gcn_body,
        grid=grid,
        in_specs=[
            pl.BlockSpec((_XBLK, v), lambda s: (jnp.where(s < nx, s, 0), 0)),
            pl.BlockSpec(
                (_BLK, n),
                lambda s: (jnp.where(s < nx, 0,
                                     jnp.where(s < nx + na, s - nx,
                                               s - nx - na)), 0)),
            pl.BlockSpec((v, hid), lambda s: (0, 0)),
            pl.BlockSpec((hid, hid), lambda s: (0, 0)),
            pl.BlockSpec((1, hid), lambda s: (0, 0)),
            pl.BlockSpec((hid, out_dim), lambda s: (0, 0)),
            pl.BlockSpec((1, out_dim), lambda s: (0, 0)),
        ],
        out_specs=pl.BlockSpec(
            (_BLK, out_dim),
            lambda s: (jnp.where(s < nx + na, 0, s - nx - na), 0)),
        out_shape=jax.ShapeDtypeStruct((n, out_dim), jnp.float32),
        scratch_shapes=[
            pltpu.VMEM((n, hid), jnp.bfloat16),
            pltpu.VMEM((n, out_dim), jnp.bfloat16),
        ],
        compiler_params=pltpu.CompilerParams(
            dimension_semantics=("arbitrary",),
        ),
    )(X_sparse, A_norm, W0, W1, b1.reshape(1, -1), W2, b2.reshape(1, -1))


def kernel(X_sparse, A_norm, W0, W1, b1, W2, b2):
    return _gcn(X_sparse, A_norm, W0, W1, b1, W2, b2)


# speedup vs baseline: 1.1427x; 1.0764x over previous
import jax
import jax.numpy as jnp
from jax.experimental import pallas as pl
from jax.experimental.pallas import tpu as pltpu

_BLK = 416
_XBLK = 2000
_QS = 254.0


def _pass1_body(x_ref, a_ref, w0_ref, w1_ref, b1_ref, w2_ref,
                p2_ref, q_ref, p1_ref):
    s = pl.program_id(0)
    nx = p1_ref.shape[0] // _XBLK

    @pl.when(s < nx)
    def _():
        h0 = jnp.maximum(
            jnp.dot(x_ref[...], w0_ref[...],
                    preferred_element_type=jnp.float32), 0.0)
        p1_ref[pl.ds(s * _XBLK, _XBLK), :] = jnp.dot(
            h0, w1_ref[...],
            preferred_element_type=jnp.float32).astype(jnp.bfloat16)

    @pl.when(s >= nx)
    def _():
        a = a_ref[...]
        q_ref[...] = jnp.round(a * _QS - 127.0).astype(jnp.int8)
        h = jnp.dot(a.astype(jnp.bfloat16), p1_ref[...],
                    preferred_element_type=jnp.float32)
        h = jnp.maximum(h + b1_ref[...], 0.0)
        p2_ref[...] = jnp.dot(h, w2_ref[...],
                              preferred_element_type=jnp.float32)


def _pass2_body(q_ref, p2_ref, b2_ref, out_ref):
    p2 = p2_ref[...]
    acc = jnp.dot(q_ref[...].astype(jnp.bfloat16), p2.astype(jnp.bfloat16),
                  preferred_element_type=jnp.float32)
    colsum = jnp.sum(p2, axis=0, keepdims=True)
    out_ref[...] = acc * (1.0 / _QS) + (127.0 / _QS) * colsum + b2_ref[...]


@jax.jit
def _gcn(X_sparse, A_norm, W0, W1, b1, W2, b2):
    n, v = X_sparse.shape
    hid = W0.shape[1]
    out_dim = W2.shape[1]
    nx = n // _XBLK
    na = pl.cdiv(n, _BLK)

    p2, q = pl.pallas_call(
        _pass1_body,
        grid=(nx + na,),
        in_specs=[
            pl.BlockSpec((_XBLK, v), lambda s: (jnp.where(s < nx, s, nx - 1), 0)),
            pl.BlockSpec((_BLK, n), lambda s: (jnp.where(s < nx, 0, s - nx), 0)),
            pl.BlockSpec((v, hid), lambda s: (0, 0)),
            pl.BlockSpec((hid, hid), lambda s: (0, 0)),
            pl.BlockSpec((1, hid), lambda s: (0, 0)),
            pl.BlockSpec((hid, out_dim), lambda s: (0, 0)),
        ],
        out_specs=[
            pl.BlockSpec((_BLK, out_dim),
                         lambda s: (jnp.where(s < nx, 0, s - nx), 0)),
            pl.BlockSpec((_BLK, n),
                         lambda s: (jnp.where(s < nx, 0, s - nx), 0)),
        ],
        out_shape=[
            jax.ShapeDtypeStruct((n, out_dim), jnp.float32),
            jax.ShapeDtypeStruct((n, n), jnp.int8),
        ],
        scratch_shapes=[pltpu.VMEM((n, hid), jnp.bfloat16)],
        compiler_params=pltpu.CompilerParams(
            dimension_semantics=("arbitrary",),
        ),
    )(X_sparse, A_norm, W0, W1, b1.reshape(1, -1), W2)

    return pl.pallas_call(
        _pass2_body,
        grid=(na,),
        in_specs=[
            pl.BlockSpec((_BLK, n), lambda s: (s, 0)),
            pl.BlockSpec((n, out_dim), lambda s: (0, 0)),
            pl.BlockSpec((1, out_dim), lambda s: (0, 0)),
        ],
        out_specs=pl.BlockSpec((_BLK, out_dim), lambda s: (s, 0)),
        out_shape=jax.ShapeDtypeStruct((n, out_dim), jnp.float32),
        compiler_params=pltpu.CompilerParams(
            dimension_semantics=("arbitrary",),
        ),
    )(q, p2, b2.reshape(1, -1))


def kernel(X_sparse, A_norm, W0, W1, b1, W2, b2):
    return _gcn(X_sparse, A_norm, W0, W1, b1, W2, b2)
